# Initial kernel scaffold; baseline (speedup 1.0000x reference)
import jax
import jax.numpy as jnp
from jax.experimental import pallas as pl


def kernel(value, depth, position, src_value_emb, depth_emb, sp_emb0, sp_emb1, sp_emb2, conv_w, conv_b):
    raise NotImplementedError("write your pallas kernel here")


# speedup vs baseline: 2.1968x; 2.1968x over previous
import functools

import jax
import jax.numpy as jnp
from jax import lax
from jax.experimental import pallas as pl
from jax.experimental.pallas import tpu as pltpu
from jax.experimental.pallas import tpu_sc as plsc

B, L = 4, 8192
C = 128
S = 8
NT = B * L
NC, NS = 2, 16
NW = NC * NS
TOK_PER_W = NT // NW
CHUNK = 128
NCHUNK = TOK_PER_W // CHUNK
NTAB = 5
ROWS = 257 + 9 + 3 * 512


def _sc_gather_sum(table, idx):
    mesh = plsc.VectorSubcoreMesh(
        core_axis_name="c", subcore_axis_name="s", num_cores=NC, num_subcores=NS
    )

    @functools.partial(
        pl.kernel,
        out_type=jax.ShapeDtypeStruct((NT, C), jnp.float32),
        mesh=mesh,
        scratch_types=[
            pltpu.VMEM((NTAB, CHUNK), jnp.int32),
            pltpu.VMEM((CHUNK, C), jnp.float32),
            pltpu.SemaphoreType.DMA,
        ],
    )
    def k(table_hbm, idx_hbm, x_hbm, idx_v, acc_v, sem):
        wid = lax.axis_index("s") * NC + lax.axis_index("c")

        def chunk_body(ci, carry):
            base = wid * TOK_PER_W + ci * CHUNK
            pltpu.sync_copy(idx_hbm.at[wid, ci], idx_v)
            pltpu.async_copy(table_hbm.at[idx_v.at[0]], acc_v, sem).wait()
            for j in range(1, NTAB):
                pltpu.async_copy(
                    table_hbm.at[idx_v.at[j]], acc_v, sem, add=True
                ).wait()
            pltpu.sync_copy(acc_v, x_hbm.at[pl.ds(base, CHUNK)])
            return carry

        lax.fori_loop(0, NCHUNK, chunk_body, 0)

    return k(table, idx)


def _conv_matmul(x2, wflat, bias2):
    rows = NT // S
    blk = 512
    grid = rows // blk

    def body(x_ref, w_ref, b_ref, o_ref):
        o_ref[...] = (
            jnp.dot(x_ref[...], w_ref[...], preferred_element_type=jnp.float32)
            + b_ref[...]
        )

    return pl.pallas_call(
        body,
        grid=(grid,),
        in_specs=[
            pl.BlockSpec((blk, S * C), lambda i: (i, 0)),
            pl.BlockSpec((S * C, C), lambda i: (0, 0)),
            pl.BlockSpec((1, C), lambda i: (0, 0)),
        ],
        out_specs=pl.BlockSpec((blk, C), lambda i: (i, 0)),
        out_shape=jax.ShapeDtypeStruct((rows, C), jnp.float32),
    )(x2, wflat, bias2)


def kernel(value, depth, position, src_value_emb, depth_emb, sp_emb0, sp_emb1,
           sp_emb2, conv_w, conv_b):
    table = jnp.concatenate(
        [
            src_value_emb.at[0].set(0.0),
            depth_emb.at[0].set(0.0),
            sp_emb0.at[0].set(0.0),
            sp_emb1.at[0].set(0.0),
            sp_emb2.at[0].set(0.0),
        ],
        axis=0,
    )
    offs = jnp.array([0, 257, 266, 778, 1290], dtype=jnp.int32)
    idx = jnp.stack(
        [
            value.reshape(-1),
            depth.reshape(-1),
            position[:, :, 0].reshape(-1),
            position[:, :, 1].reshape(-1),
            position[:, :, 2].reshape(-1),
        ],
        axis=0,
    ) + offs[:, None]
    idx = idx.reshape(NTAB, NW, NCHUNK, CHUNK).transpose(1, 2, 0, 3)

    x = _sc_gather_sum(table, idx)
    x2 = x.reshape(NT // S, S * C)
    wflat = conv_w.transpose(2, 1, 0).reshape(S * C, C)
    y = _conv_matmul(x2, wflat, conv_b.reshape(1, C))
    return y.reshape(B, NT // (S * B), C)


# speedup vs baseline: 2.2994x; 1.0467x over previous
import functools

import jax
import jax.numpy as jnp
from jax import lax
from jax.experimental import pallas as pl
from jax.experimental.pallas import tpu as pltpu
from jax.experimental.pallas import tpu_sc as plsc

B, L = 4, 8192
C = 128
S = 8
NT = B * L
NC, NS = 2, 16
NW = NC * NS
TOK_PER_W = NT // NW
CHUNK = 128
NCHUNK = TOK_PER_W // CHUNK
NTAB = 5
ROWS = 257 + 9 + 3 * 512


def _sc_gather_sum(table, idx):
    mesh = plsc.VectorSubcoreMesh(
        core_axis_name="c", subcore_axis_name="s", num_cores=NC, num_subcores=NS
    )

    @functools.partial(
        pl.kernel,
        out_type=jax.ShapeDtypeStruct((NT, C), jnp.float32),
        mesh=mesh,
        scratch_types=[
            pltpu.VMEM((NTAB, CHUNK), jnp.int32),
            pltpu.VMEM((NTAB, CHUNK), jnp.int32),
            pltpu.VMEM((CHUNK, C), jnp.float32),
            pltpu.VMEM((CHUNK, C), jnp.float32),
            pltpu.SemaphoreType.DMA,
            pltpu.SemaphoreType.DMA,
            pltpu.SemaphoreType.DMA,
            pltpu.SemaphoreType.DMA,
            pltpu.SemaphoreType.DMA,
            pltpu.SemaphoreType.DMA,
        ],
    )
    def k(table_hbm, idx_hbm, x_hbm, idx_a, idx_b, acc_a, acc_b,
          ga, gb, aa, ab, wa, wb):
        wid = lax.axis_index("s") * NC + lax.axis_index("c")

        def pair_body(i2, carry):
            c0 = 2 * i2
            base0 = wid * TOK_PER_W + c0 * CHUNK
            base1 = base0 + CHUNK

            pltpu.sync_copy(idx_hbm.at[wid, c0], idx_a)

            @pl.when(i2 > 0)
            def _():
                pltpu.make_async_copy(
                    acc_a, x_hbm.at[pl.ds(base0 - 2 * CHUNK, CHUNK)], wa
                ).wait()

            first_a = pltpu.async_copy(table_hbm.at[idx_a.at[0]], acc_a, ga)

            pltpu.sync_copy(idx_hbm.at[wid, c0 + 1], idx_b)

            @pl.when(i2 > 0)
            def _():
                pltpu.make_async_copy(
                    acc_b, x_hbm.at[pl.ds(base1 - 2 * CHUNK, CHUNK)], wb
                ).wait()

            first_b = pltpu.async_copy(table_hbm.at[idx_b.at[0]], acc_b, gb)

            first_a.wait()
            adds_a = [
                pltpu.async_copy(table_hbm.at[idx_a.at[j]], acc_a, aa, add=True)
                for j in range(1, NTAB)
            ]
            first_b.wait()
            adds_b = [
                pltpu.async_copy(table_hbm.at[idx_b.at[j]], acc_b, ab, add=True)
                for j in range(1, NTAB)
            ]
            for d in adds_a:
                d.wait()
            pltpu.async_copy(acc_a, x_hbm.at[pl.ds(base0, CHUNK)], wa)
            for d in adds_b:
                d.wait()
            pltpu.async_copy(acc_b, x_hbm.at[pl.ds(base1, CHUNK)], wb)
            return carry

        lax.fori_loop(0, NCHUNK // 2, pair_body, 0)
        last0 = wid * TOK_PER_W + (NCHUNK - 2) * CHUNK
        pltpu.make_async_copy(acc_a, x_hbm.at[pl.ds(last0, CHUNK)], wa).wait()
        pltpu.make_async_copy(
            acc_b, x_hbm.at[pl.ds(last0 + CHUNK, CHUNK)], wb
        ).wait()

    return k(table, idx)


def _conv_matmul(x2, wflat, bias2):
    rows = NT // S
    blk = 512
    grid = rows // blk

    def body(x_ref, w_ref, b_ref, o_ref):
        o_ref[...] = (
            jnp.dot(x_ref[...], w_ref[...], preferred_element_type=jnp.float32)
            + b_ref[...]
        )

    return pl.pallas_call(
        body,
        grid=(grid,),
        in_specs=[
            pl.BlockSpec((blk, S * C), lambda i: (i, 0)),
            pl.BlockSpec((S * C, C), lambda i: (0, 0)),
            pl.BlockSpec((1, C), lambda i: (0, 0)),
        ],
        out_specs=pl.BlockSpec((blk, C), lambda i: (i, 0)),
        out_shape=jax.ShapeDtypeStruct((rows, C), jnp.float32),
    )(x2, wflat, bias2)


def kernel(value, depth, position, src_value_emb, depth_emb, sp_emb0, sp_emb1,
           sp_emb2, conv_w, conv_b):
    table = jnp.concatenate(
        [
            src_value_emb.at[0].set(0.0),
            depth_emb.at[0].set(0.0),
            sp_emb0.at[0].set(0.0),
            sp_emb1.at[0].set(0.0),
            sp_emb2.at[0].set(0.0),
        ],
        axis=0,
    )
    offs = jnp.array([0, 257, 266, 778, 1290], dtype=jnp.int32)
    idx = jnp.stack(
        [
            value.reshape(-1),
            depth.reshape(-1),
            position[:, :, 0].reshape(-1),
            position[:, :, 1].reshape(-1),
            position[:, :, 2].reshape(-1),
        ],
        axis=0,
    ) + offs[:, None]
    idx = idx.reshape(NTAB, NW, NCHUNK, CHUNK).transpose(1, 2, 0, 3)

    x = _sc_gather_sum(table, idx)
    x2 = x.reshape(NT // S, S * C)
    wflat = conv_w.transpose(2, 1, 0).reshape(S * C, C)
    y = _conv_matmul(x2, wflat, conv_b.reshape(1, C))
    return y.reshape(B, NT // (S * B), C)


# speedup vs baseline: 6.4660x; 2.8120x over previous
import functools

import jax
import jax.numpy as jnp
from jax import lax
from jax.experimental import pallas as pl
from jax.experimental.pallas import tpu as pltpu
from jax.experimental.pallas import tpu_sc as plsc

B, L = 4, 8192
C = 128
S = 8
NT = B * L
NC, NS = 2, 16
NW = NC * NS
TOK_PER_W = NT // NW
CHUNK = 128
NCHUNK = TOK_PER_W // CHUNK
NTAB = 5
NBUF = 4
ROWS = 257 + 9 + 3 * 512
ROWS_PAD = 1920
RPT = ROWS_PAD // NS


def _sc_gather_sum(table, idx):
    mesh = plsc.VectorSubcoreMesh(
        core_axis_name="c", subcore_axis_name="s", num_cores=NC, num_subcores=NS
    )

    @functools.partial(
        pl.kernel,
        out_type=jax.ShapeDtypeStruct((NT, C), jnp.float32),
        mesh=mesh,
        scratch_types=[
            pltpu.VMEM_SHARED((ROWS_PAD, C), jnp.float32),
            pltpu.VMEM((RPT, C), jnp.float32),
            pltpu.VMEM((NCHUNK, NTAB, CHUNK), jnp.int32),
            [pltpu.VMEM((CHUNK, C), jnp.float32) for _ in range(NBUF)],
            [pltpu.SemaphoreType.DMA for _ in range(NBUF)],
            [pltpu.SemaphoreType.DMA for _ in range(NBUF)],
            [pltpu.SemaphoreType.DMA for _ in range(NBUF)],
        ],
    )
    def k(table_hbm, idx_hbm, x_hbm, table_sp, stage_v, idx_v, accs, gsems,
          asems, wsems):
        sid = lax.axis_index("s")
        wid = sid * NC + lax.axis_index("c")
        tbase = wid * TOK_PER_W

        pltpu.sync_copy(table_hbm.at[pl.ds(sid * RPT, RPT)], stage_v)
        pltpu.sync_copy(stage_v, table_sp.at[pl.ds(sid * RPT, RPT)])
        pltpu.sync_copy(idx_hbm.at[wid], idx_v)
        plsc.subcore_barrier()

        for b in range(NBUF):
            pltpu.async_copy(table_sp.at[idx_v.at[b, 0]], accs[b], gsems[b])

        def body(i4, carry):
            for b in range(NBUF):
                c = i4 * NBUF + b
                pltpu.make_async_copy(
                    table_sp.at[idx_v.at[c, 0]], accs[b], gsems[b]
                ).wait()
                for j in range(1, NTAB):
                    pltpu.async_copy(
                        table_sp.at[idx_v.at[c, j]], accs[b], asems[b],
                        add=True,
                    )
            for b in range(NBUF):
                c = i4 * NBUF + b
                for j in range(1, NTAB):
                    pltpu.make_async_copy(
                        table_sp.at[idx_v.at[c, j]], accs[b], asems[b]
                    ).wait()
                pltpu.async_copy(
                    accs[b], x_hbm.at[pl.ds(tbase + c * CHUNK, CHUNK)],
                    wsems[b],
                )

                @pl.when(c + NBUF < NCHUNK)
                def _(b=b, c=c):
                    pltpu.make_async_copy(
                        accs[b], x_hbm.at[pl.ds(tbase + c * CHUNK, CHUNK)],
                        wsems[b],
                    ).wait()
                    pltpu.async_copy(
                        table_sp.at[idx_v.at[c + NBUF, 0]], accs[b], gsems[b]
                    )
            return carry

        lax.fori_loop(0, NCHUNK // NBUF, body, 0)
        for b in range(NBUF):
            c = NCHUNK - NBUF + b
            pltpu.make_async_copy(
                accs[b], x_hbm.at[pl.ds(tbase + c * CHUNK, CHUNK)], wsems[b]
            ).wait()

    return k(table, idx)


def _conv_matmul(x2, wflat, bias2):
    rows = NT // S
    blk = 512
    grid = rows // blk

    def body(x_ref, w_ref, b_ref, o_ref):
        o_ref[...] = (
            jnp.dot(x_ref[...], w_ref[...], preferred_element_type=jnp.float32)
            + b_ref[...]
        )

    return pl.pallas_call(
        body,
        grid=(grid,),
        in_specs=[
            pl.BlockSpec((blk, S * C), lambda i: (i, 0)),
            pl.BlockSpec((S * C, C), lambda i: (0, 0)),
            pl.BlockSpec((1, C), lambda i: (0, 0)),
        ],
        out_specs=pl.BlockSpec((blk, C), lambda i: (i, 0)),
        out_shape=jax.ShapeDtypeStruct((rows, C), jnp.float32),
    )(x2, wflat, bias2)


def kernel(value, depth, position, src_value_emb, depth_emb, sp_emb0, sp_emb1,
           sp_emb2, conv_w, conv_b):
    table = jnp.concatenate(
        [
            src_value_emb.at[0].set(0.0),
            depth_emb.at[0].set(0.0),
            sp_emb0.at[0].set(0.0),
            sp_emb1.at[0].set(0.0),
            sp_emb2.at[0].set(0.0),
        ],
        axis=0,
    )
    table = jnp.pad(table, ((0, ROWS_PAD - ROWS), (0, 0)))
    offs = jnp.array([0, 257, 266, 778, 1290], dtype=jnp.int32)
    idx = jnp.stack(
        [
            value.reshape(-1),
            depth.reshape(-1),
            position[:, :, 0].reshape(-1),
            position[:, :, 1].reshape(-1),
            position[:, :, 2].reshape(-1),
        ],
        axis=0,
    ) + offs[:, None]
    idx = idx.reshape(NTAB, NW, NCHUNK, CHUNK).transpose(1, 2, 0, 3)

    x = _sc_gather_sum(table, idx)
    x2 = x.reshape(NT // S, S * C)
    wflat = conv_w.transpose(2, 1, 0).reshape(S * C, C)
    y = _conv_matmul(x2, wflat, conv_b.reshape(1, C))
    return y.reshape(B, NT // (S * B), C)


# speedup vs baseline: 7.1472x; 1.1054x over previous
import functools

import jax
import jax.numpy as jnp
from jax import lax
from jax.experimental import pallas as pl
from jax.experimental.pallas import tpu as pltpu
from jax.experimental.pallas import tpu_sc as plsc

B, L = 4, 8192
C = 128
S = 8
NT = B * L
NC, NS = 2, 16
NW = NC * NS
TOK_PER_W = NT // NW
CHUNK = 128
NCHUNK = TOK_PER_W // CHUNK
NTAB = 4
NBUF = 4
ROWS = 257 * 9 + 3 * 512
ROWS_PAD = 4096
RPT = ROWS_PAD // NS


def _sc_gather_sum(table, idx):
    mesh = plsc.VectorSubcoreMesh(
        core_axis_name="c", subcore_axis_name="s", num_cores=NC, num_subcores=NS
    )

    @functools.partial(
        pl.kernel,
        out_type=jax.ShapeDtypeStruct((NT, C), jnp.float32),
        mesh=mesh,
        scratch_types=[
            pltpu.VMEM_SHARED((ROWS_PAD, C), jnp.float32),
            pltpu.VMEM((NCHUNK, NTAB, CHUNK), jnp.int32),
            [pltpu.VMEM((CHUNK, C), jnp.float32) for _ in range(NBUF)],
            [pltpu.SemaphoreType.DMA for _ in range(NBUF)],
            [pltpu.SemaphoreType.DMA for _ in range(NBUF)],
            [pltpu.SemaphoreType.DMA for _ in range(NBUF)],
        ],
    )
    def k(table_hbm, idx_hbm, x_hbm, table_sp, idx_v, accs, gsems,
          asems, wsems):
        sid = lax.axis_index("s")
        wid = sid * NC + lax.axis_index("c")
        tbase = wid * TOK_PER_W

        for p in range(RPT // CHUNK):
            r0 = sid * RPT + p * CHUNK
            pltpu.sync_copy(table_hbm.at[pl.ds(r0, CHUNK)], accs[p % NBUF])
            pltpu.sync_copy(accs[p % NBUF], table_sp.at[pl.ds(r0, CHUNK)])
        pltpu.sync_copy(idx_hbm.at[wid], idx_v)
        plsc.subcore_barrier()

        for b in range(NBUF):
            pltpu.async_copy(table_sp.at[idx_v.at[b, 0]], accs[b], gsems[b])

        def body(i4, carry):
            for b in range(NBUF):
                c = i4 * NBUF + b
                pltpu.make_async_copy(
                    table_sp.at[idx_v.at[c, 0]], accs[b], gsems[b]
                ).wait()
                for j in range(1, NTAB):
                    pltpu.async_copy(
                        table_sp.at[idx_v.at[c, j]], accs[b], asems[b],
                        add=True,
                    )
            for b in range(NBUF):
                c = i4 * NBUF + b
                for j in range(1, NTAB):
                    pltpu.make_async_copy(
                        table_sp.at[idx_v.at[c, j]], accs[b], asems[b]
                    ).wait()
                pltpu.async_copy(
                    accs[b], x_hbm.at[pl.ds(tbase + c * CHUNK, CHUNK)],
                    wsems[b],
                )

                @pl.when(c + NBUF < NCHUNK)
                def _(b=b, c=c):
                    pltpu.make_async_copy(
                        accs[b], x_hbm.at[pl.ds(tbase + c * CHUNK, CHUNK)],
                        wsems[b],
                    ).wait()
                    pltpu.async_copy(
                        table_sp.at[idx_v.at[c + NBUF, 0]], accs[b], gsems[b]
                    )
            return carry

        lax.fori_loop(0, NCHUNK // NBUF, body, 0)
        for b in range(NBUF):
            c = NCHUNK - NBUF + b
            pltpu.make_async_copy(
                accs[b], x_hbm.at[pl.ds(tbase + c * CHUNK, CHUNK)], wsems[b]
            ).wait()

    return k(table, idx)


def _conv_matmul(x2, wflat, bias2):
    rows = NT // S
    blk = 512
    grid = rows // blk

    def body(x_ref, w_ref, b_ref, o_ref):
        o_ref[...] = (
            jnp.dot(x_ref[...], w_ref[...], preferred_element_type=jnp.float32)
            + b_ref[...]
        )

    return pl.pallas_call(
        body,
        grid=(grid,),
        in_specs=[
            pl.BlockSpec((blk, S * C), lambda i: (i, 0)),
            pl.BlockSpec((S * C, C), lambda i: (0, 0)),
            pl.BlockSpec((1, C), lambda i: (0, 0)),
        ],
        out_specs=pl.BlockSpec((blk, C), lambda i: (i, 0)),
        out_shape=jax.ShapeDtypeStruct((rows, C), jnp.float32),
    )(x2, wflat, bias2)


def kernel(value, depth, position, src_value_emb, depth_emb, sp_emb0, sp_emb1,
           sp_emb2, conv_w, conv_b):
    vd = (src_value_emb.at[0].set(0.0)[:, None, :]
          + depth_emb.at[0].set(0.0)[None, :, :]).reshape(257 * 9, C)
    table = jnp.concatenate(
        [
            vd,
            sp_emb0.at[0].set(0.0),
            sp_emb1.at[0].set(0.0),
            sp_emb2.at[0].set(0.0),
        ],
        axis=0,
    )
    table = jnp.pad(table, ((0, ROWS_PAD - ROWS), (0, 0)))
    offs = jnp.array([0, 2313, 2825, 3337], dtype=jnp.int32)
    idx = jnp.stack(
        [
            value.reshape(-1) * 9 + depth.reshape(-1),
            position[:, :, 0].reshape(-1),
            position[:, :, 1].reshape(-1),
            position[:, :, 2].reshape(-1),
        ],
        axis=0,
    ) + offs[:, None]
    idx = idx.reshape(NTAB, NW, NCHUNK, CHUNK).transpose(1, 2, 0, 3)

    x = _sc_gather_sum(table, idx)
    x2 = x.reshape(NT // S, S * C)
    wflat = conv_w.transpose(2, 1, 0).reshape(S * C, C)
    y = _conv_matmul(x2, wflat, conv_b.reshape(1, C))
    return y.reshape(B, NT // (S * B), C)
